# Initial kernel scaffold; baseline (speedup 1.0000x reference)
import jax
import jax.numpy as jnp
from jax.experimental import pallas as pl


def kernel(nodes, edges, timestep, node_table, edge_table, time_table):
    raise NotImplementedError("write your pallas kernel here")


# speedup vs baseline: 6.6302x; 6.6302x over previous
import functools

import jax
import jax.numpy as jnp
from jax import lax
from jax.experimental import pallas as pl
from jax.experimental.pallas import tpu as pltpu

N = 1024
D = 64
NODE_STATES = 32
EDGE_STATES = 8
TIME_ROWS = 1000

ROWS_PER_BLOCK = 8


def _edge_kernel(e_ref, tbl_ref, out_ref):
    e = e_ref[:]
    R = e.shape[0]
    states = lax.broadcasted_iota(jnp.int32, (R, N, EDGE_STATES), 2)
    oh = (e[:, :, None] == states).astype(jnp.float32)
    oh2 = oh.reshape(R * N, EDGE_STATES)
    res = jnp.dot(oh2, tbl_ref[:], preferred_element_type=jnp.float32)
    out_ref[:] = res.reshape(R, N, D)


def _small_kernel(t_ref, nodes_ref, ntbl_ref, ttbl_ref, nout_ref, tout_ref):
    nodes = nodes_ref[:]
    st = lax.broadcasted_iota(jnp.int32, (8, 128, NODE_STATES), 2)
    oh = (nodes[:, :, None] == st).astype(jnp.float32)
    res = jnp.dot(oh.reshape(N, NODE_STATES), ntbl_ref[:],
                  preferred_element_type=jnp.float32)
    nout_ref[:] = res
    t = t_ref[0]
    trows = lax.broadcasted_iota(jnp.int32, (8, TIME_ROWS), 1)
    toh = (trows == t).astype(jnp.float32)
    tres = jnp.dot(toh, ttbl_ref[:], preferred_element_type=jnp.float32)
    tout_ref[:] = tres[0:1, :]


@jax.jit
def kernel(nodes, edges, timestep, node_table, edge_table, time_table):
    edge_emb = pl.pallas_call(
        _edge_kernel,
        grid=(N // ROWS_PER_BLOCK,),
        in_specs=[
            pl.BlockSpec((ROWS_PER_BLOCK, N), lambda i: (i, 0)),
            pl.BlockSpec((EDGE_STATES, D), lambda i: (0, 0)),
        ],
        out_specs=pl.BlockSpec((ROWS_PER_BLOCK, N, D), lambda i: (i, 0, 0)),
        out_shape=jax.ShapeDtypeStruct((N, N, D), jnp.float32),
    )(edges, edge_table)

    t_arr = jnp.asarray(timestep, dtype=jnp.int32).reshape(1)
    nodes2d = nodes.astype(jnp.int32).reshape(8, 128)
    node_emb, time_row = pl.pallas_call(
        _small_kernel,
        in_specs=[
            pl.BlockSpec(memory_space=pltpu.SMEM),
            pl.BlockSpec((8, 128), lambda: (0, 0)),
            pl.BlockSpec((NODE_STATES, D), lambda: (0, 0)),
            pl.BlockSpec((TIME_ROWS, D), lambda: (0, 0)),
        ],
        out_specs=[
            pl.BlockSpec((N, D), lambda: (0, 0)),
            pl.BlockSpec((1, D), lambda: (0, 0)),
        ],
        out_shape=[
            jax.ShapeDtypeStruct((N, D), jnp.float32),
            jax.ShapeDtypeStruct((1, D), jnp.float32),
        ],
    )(t_arr, nodes2d, node_table, time_table)

    return (node_emb, edge_emb, time_row[0])
